# baseline (speedup 1.0000x reference)
import jax
import jax.numpy as jnp
from jax import lax
from jax.experimental import pallas as pl
from jax.experimental.pallas import tpu as pltpu
from jax.experimental.pallas import tpu_sc as plsc

H = W = 512
P = H * W
C = 16
TH, TW = 1024, 512
NC, NS, L = 2, 16, 16
NW = NC * NS
PPW = P // NW
CH = 128
NCHUNK = PPW // CH
G = CH // L

LEVELS = ((0, 512), (512, 256), (768, 128))
L3_OFF, L3_W = 896, 64
L3_PAD = L3_W + 1


def _bcast(wv, i):
    s = lax.squeeze(lax.slice(wv, (i,), (i + 1,)), (0,))
    return jnp.broadcast_to(s, (L,))


def _body(ux_hbm, uy_hbm, table_hbm, out_hbm,
          ux_v, uy_v, l3_v,
          i00a, i01a, i10a, i11a, wxa, wya, t00a, t01a, t10a, t11a,
          i00b, i01b, i10b, i11b, wxb, wyb, t00b, t01b, t10b, t11b,
          acc_v, sem_a, sem_b, sem_s):
    bufs = ((i00a, i01a, i10a, i11a, wxa, wya, t00a, t01a, t10a, t11a, sem_a),
            (i00b, i01b, i10b, i11b, wxb, wyb, t00b, t01b, t10b, t11b, sem_b))

    cid = lax.axis_index("c")
    sid = lax.axis_index("s")
    wid = sid * NC + cid
    pbase = wid * PPW

    def stage_grp(yy, c2):
        for y4 in range(16):
            y = yy * 16 + y4
            pltpu.async_copy(
                table_hbm.at[pl.ds((L3_OFF + y) * TW, L3_W)],
                l3_v.at[pl.ds(y * L3_W, L3_W)], sem_s)
        for y4 in range(16):
            y = yy * 16 + y4
            pltpu.make_async_copy(
                table_hbm.at[pl.ds((L3_OFF + y) * TW, L3_W)],
                l3_v.at[pl.ds(y * L3_W, L3_W)], sem_s).wait()
        return c2

    lax.fori_loop(0, 4, stage_grp, 0)

    def zpad(j, c2):
        l3_v[L3_W * L3_W + j] = jnp.zeros((L,), jnp.float32)
        return c2

    lax.fori_loop(0, L3_PAD, zpad, 0)

    pltpu.sync_copy(ux_hbm.at[pl.ds(pbase, PPW)], ux_v)
    pltpu.sync_copy(uy_hbm.at[pl.ds(pbase, PPW)], uy_v)

    def coords(s, w):
        gx = ux_v[pl.ds(s, L)]
        gy = uy_v[pl.ds(s, L)]
        x = ((gx + 1.0) * w - 1.0) * 0.5
        y = ((gy + 1.0) * w - 1.0) * 0.5
        x = jnp.minimum(jnp.maximum(x, 0.0), w - 1.0)
        y = jnp.minimum(jnp.maximum(y, 0.0), w - 1.0)
        xi = x.astype(jnp.int32)
        yi = y.astype(jnp.int32)
        fx = x - xi.astype(jnp.float32)
        fy = y - yi.astype(jnp.float32)
        return xi, yi, fx, fy

    def build_and_fire(bset, base, lvl):
        i00, i01, i10, i11, wx, wy, t00, t01, t10, t11, sem = bset
        off_y, w = LEVELS[lvl]

        def build_g(g, c2):
            xi, yi, fx, fy = coords(base + g * L, w)
            gs = pl.ds(g * L, L)
            b = (yi + off_y) * TW + xi
            i00[gs] = b
            i01[gs] = b + 1
            i10[gs] = b + TW
            i11[gs] = b + (TW + 1)
            wx[gs] = fx
            wy[gs] = fy
            return c2

        lax.fori_loop(0, G, build_g, 0)
        pltpu.async_copy(table_hbm.at[i00], t00, sem)
        pltpu.async_copy(table_hbm.at[i01], t01, sem)
        pltpu.async_copy(table_hbm.at[i10], t10, sem)
        pltpu.async_copy(table_hbm.at[i11], t11, sem)

    def wait_and_combine(bset, first):
        i00, i01, i10, i11, wx, wy, t00, t01, t10, t11, sem = bset
        pltpu.make_async_copy(table_hbm.at[i00], t00, sem).wait()
        pltpu.make_async_copy(table_hbm.at[i01], t01, sem).wait()
        pltpu.make_async_copy(table_hbm.at[i10], t10, sem).wait()
        pltpu.make_async_copy(table_hbm.at[i11], t11, sem).wait()

        def comb_g(g, c2):
            gs = pl.ds(g * L, L)
            wxv = wx[gs]
            wyv = wy[gs]
            for i in range(L):
                pix = g * L + i
                fx = _bcast(wxv, i)
                fy = _bcast(wyv, i)
                a00 = t00[pix]
                a01 = t01[pix]
                a10 = t10[pix]
                a11 = t11[pix]
                top = a00 + fx * (a01 - a00)
                bot = a10 + fx * (a11 - a10)
                a = top + fy * (bot - top)
                if first:
                    acc_v[pix] = a
                else:
                    plsc.addupdate(acc_v.at[pix], a)
            return c2

        lax.fori_loop(0, G, comb_g, 0)

    def combine_l3(base):
        def l3_g(g, c2):
            xi, yi, fxv, fyv = coords(base + g * L, L3_W)
            b = yi * L3_W + xi
            for i in range(L):
                pix = g * L + i
                bi = lax.squeeze(lax.slice(b, (i,), (i + 1,)), (0,))
                fx = _bcast(fxv, i)
                fy = _bcast(fyv, i)
                a00 = l3_v[bi]
                a01 = l3_v[bi + 1]
                a10 = l3_v[bi + L3_W]
                a11 = l3_v[bi + (L3_W + 1)]
                top = a00 + fx * (a01 - a00)
                bot = a10 + fx * (a11 - a10)
                a = top + fy * (bot - top)
                acc_v[pix] = a
            return c2

        lax.fori_loop(0, G, l3_g, 0)

    def write_chunk(cbase):
        pltpu.sync_copy(acc_v, out_hbm.at[pl.ds(pbase + cbase, CH)])

    build_and_fire(bufs[0], 0, 0)

    def pair_body(cp, carry):
        cb0 = cp * (2 * CH)
        cb1 = cb0 + CH
        build_and_fire(bufs[1], cb0, 1)
        combine_l3(cb0)
        wait_and_combine(bufs[0], False)
        build_and_fire(bufs[0], cb0, 2)
        wait_and_combine(bufs[1], False)
        build_and_fire(bufs[1], cb1, 0)
        wait_and_combine(bufs[0], False)
        write_chunk(cb0)
        build_and_fire(bufs[0], cb1, 1)
        combine_l3(cb1)
        wait_and_combine(bufs[1], False)
        build_and_fire(bufs[1], cb1, 2)
        wait_and_combine(bufs[0], False)

        @pl.when(cp < (NCHUNK // 2) - 1)
        def _prefetch():
            build_and_fire(bufs[0], cb1 + CH, 0)

        wait_and_combine(bufs[1], False)
        write_chunk(cb1)
        return carry

    lax.fori_loop(0, NCHUNK // 2, pair_body, 0)


def kernel(uv_inputs, texture_id, data):
    tex = lax.dynamic_index_in_dim(data, texture_id, axis=0, keepdims=False)
    table = jnp.transpose(tex, (1, 2, 0)).reshape(TH * TW, C)
    ux = uv_inputs[0, 0].reshape(P)
    uy = uv_inputs[0, 1].reshape(P)

    mesh = plsc.VectorSubcoreMesh(core_axis_name="c", subcore_axis_name="s")
    dbl = [
        pltpu.VMEM((CH,), jnp.int32),
        pltpu.VMEM((CH,), jnp.int32),
        pltpu.VMEM((CH,), jnp.int32),
        pltpu.VMEM((CH,), jnp.int32),
        pltpu.VMEM((CH,), jnp.float32),
        pltpu.VMEM((CH,), jnp.float32),
        pltpu.VMEM((CH, C), jnp.float32),
        pltpu.VMEM((CH, C), jnp.float32),
        pltpu.VMEM((CH, C), jnp.float32),
        pltpu.VMEM((CH, C), jnp.float32),
    ]
    run = pl.kernel(
        _body,
        out_type=jax.ShapeDtypeStruct((P, C), jnp.float32),
        mesh=mesh,
        scratch_types=(
            [pltpu.VMEM((PPW,), jnp.float32),
             pltpu.VMEM((PPW,), jnp.float32),
             pltpu.VMEM((L3_W * L3_W + L3_PAD, C), jnp.float32)]
            + dbl + dbl
            + [pltpu.VMEM((CH, C), jnp.float32),
               pltpu.SemaphoreType.DMA, pltpu.SemaphoreType.DMA,
               pltpu.SemaphoreType.DMA]
        ),
        compiler_params=pltpu.CompilerParams(use_tc_tiling_on_sc=False),
    )
    out = run(ux, uy, table)
    return jnp.transpose(out.reshape(1, H, W, C), (0, 3, 1, 2))


# speedup vs baseline: 1.0765x; 1.0765x over previous
import jax
import jax.numpy as jnp
from jax import lax
from jax.experimental import pallas as pl
from jax.experimental.pallas import tpu as pltpu
from jax.experimental.pallas import tpu_sc as plsc

H = W = 512
P = H * W
C = 16
TH, TW = 1024, 512
NC, NS, L = 2, 16, 16
NW = NC * NS
PPW = P // NW
CH = 128
NCHUNK = PPW // CH
G = CH // L

LEVELS = ((0, 512), (512, 256), (768, 128), (896, 64))


def _body(ux_hbm, uy_hbm, table_hbm, out_hbm,
          ux_v, uy_v,
          i00a, i01a, i10a, i11a, wxa, wya, t00a, t01a, t10a, t11a,
          i00b, i01b, i10b, i11b, wxb, wyb, t00b, t01b, t10b, t11b,
          acc_v, sem_a, sem_b):
    bufs = ((i00a, i01a, i10a, i11a, wxa, wya, t00a, t01a, t10a, t11a, sem_a),
            (i00b, i01b, i10b, i11b, wxb, wyb, t00b, t01b, t10b, t11b, sem_b))

    cid = lax.axis_index("c")
    sid = lax.axis_index("s")
    wid = sid * NC + cid
    pbase = wid * PPW
    pltpu.sync_copy(ux_hbm.at[pl.ds(pbase, PPW)], ux_v)
    pltpu.sync_copy(uy_hbm.at[pl.ds(pbase, PPW)], uy_v)

    def build_and_fire(bset, base, lvl):
        i00, i01, i10, i11, wx, wy, t00, t01, t10, t11, sem = bset
        off_y, w = LEVELS[lvl]

        @plsc.parallel_loop(0, G)
        def build_g(g):
            gx = ux_v[pl.ds(base + g * L, L)]
            gy = uy_v[pl.ds(base + g * L, L)]
            x = ((gx + 1.0) * w - 1.0) * 0.5
            y = ((gy + 1.0) * w - 1.0) * 0.5
            x = jnp.minimum(jnp.maximum(x, 0.0), w - 1.0)
            y = jnp.minimum(jnp.maximum(y, 0.0), w - 1.0)
            xi = x.astype(jnp.int32)
            yi = y.astype(jnp.int32)
            gs = pl.ds(g * L, L)
            b = (yi + off_y) * TW + xi
            i00[gs] = b
            i01[gs] = b + 1
            i10[gs] = b + TW
            i11[gs] = b + (TW + 1)
            wx[gs] = x - xi.astype(jnp.float32)
            wy[gs] = y - yi.astype(jnp.float32)

        pltpu.async_copy(table_hbm.at[i00], t00, sem)
        pltpu.async_copy(table_hbm.at[i01], t01, sem)
        pltpu.async_copy(table_hbm.at[i10], t10, sem)
        pltpu.async_copy(table_hbm.at[i11], t11, sem)

    def wait_and_combine(bset, lvl):
        i00, i01, i10, i11, wx, wy, t00, t01, t10, t11, sem = bset
        pltpu.make_async_copy(table_hbm.at[i00], t00, sem).wait()
        pltpu.make_async_copy(table_hbm.at[i01], t01, sem).wait()
        pltpu.make_async_copy(table_hbm.at[i10], t10, sem).wait()
        pltpu.make_async_copy(table_hbm.at[i11], t11, sem).wait()

        @plsc.parallel_loop(0, G)
        def comb_g(g):
            gs = pl.ds(g * L, L)
            wxv = wx[gs]
            wyv = wy[gs]

            def bcast(wv, i):
                s = lax.squeeze(lax.slice(wv, (i,), (i + 1,)), (0,))
                return jnp.broadcast_to(s, (L,))

            for i in range(L):
                pix = g * L + i
                fx = bcast(wxv, i)
                fy = bcast(wyv, i)
                a00 = t00[pix]
                a01 = t01[pix]
                a10 = t10[pix]
                a11 = t11[pix]
                top = a00 + fx * (a01 - a00)
                bot = a10 + fx * (a11 - a10)
                a = top + fy * (bot - top)
                if lvl == 0:
                    acc_v[pix] = a
                else:
                    plsc.addupdate(acc_v.at[pix], a)

    build_and_fire(bufs[0], 0, 0)

    def chunk_body(ch, carry):
        cbase = ch * CH
        for lvl in range(4):
            p, q = lvl % 2, (lvl + 1) % 2
            if lvl < 3:
                build_and_fire(bufs[q], cbase, lvl + 1)
            else:
                @pl.when(ch < NCHUNK - 1)
                def _prefetch():
                    build_and_fire(bufs[q], cbase + CH, 0)
            wait_and_combine(bufs[p], lvl)
        pltpu.sync_copy(acc_v, out_hbm.at[pl.ds(pbase + cbase, CH)])
        return carry

    lax.fori_loop(0, NCHUNK, chunk_body, 0)


def kernel(uv_inputs, texture_id, data):
    tex = lax.dynamic_index_in_dim(data, texture_id, axis=0, keepdims=False)
    table = jnp.transpose(tex, (1, 2, 0)).reshape(TH * TW, C)
    ux = uv_inputs[0, 0].reshape(P)
    uy = uv_inputs[0, 1].reshape(P)

    mesh = plsc.VectorSubcoreMesh(core_axis_name="c", subcore_axis_name="s")
    dbl = [
        pltpu.VMEM((CH,), jnp.int32),
        pltpu.VMEM((CH,), jnp.int32),
        pltpu.VMEM((CH,), jnp.int32),
        pltpu.VMEM((CH,), jnp.int32),
        pltpu.VMEM((CH,), jnp.float32),
        pltpu.VMEM((CH,), jnp.float32),
        pltpu.VMEM((CH, C), jnp.float32),
        pltpu.VMEM((CH, C), jnp.float32),
        pltpu.VMEM((CH, C), jnp.float32),
        pltpu.VMEM((CH, C), jnp.float32),
    ]
    run = pl.kernel(
        _body,
        out_type=jax.ShapeDtypeStruct((P, C), jnp.float32),
        mesh=mesh,
        scratch_types=(
            [pltpu.VMEM((PPW,), jnp.float32), pltpu.VMEM((PPW,), jnp.float32)]
            + dbl + dbl
            + [pltpu.VMEM((CH, C), jnp.float32),
               pltpu.SemaphoreType.DMA, pltpu.SemaphoreType.DMA]
        ),
        compiler_params=pltpu.CompilerParams(use_tc_tiling_on_sc=False),
    )
    out = run(ux, uy, table)
    return jnp.transpose(out.reshape(1, H, W, C), (0, 3, 1, 2))


# speedup vs baseline: 1.1695x; 1.0864x over previous
import jax
import jax.numpy as jnp
from jax import lax
from jax.experimental import pallas as pl
from jax.experimental.pallas import tpu as pltpu
from jax.experimental.pallas import tpu_sc as plsc

H = W = 512
P = H * W
C = 16
TH, TW = 1024, 512
NC, NS, L = 2, 16, 16
NW = NC * NS
PPW = P // NW
CH = 128
NCHUNK = PPW // CH
G = CH // L

LEVELS = ((0, 512), (512, 256), (768, 128), (896, 64))


def _body(ux_hbm, uy_hbm, table_hbm, out_hbm, ux_v, uy_v, *rest):
    bufs = tuple(rest[k * 11:(k + 1) * 11] for k in range(4))
    acc_v = rest[44]

    cid = lax.axis_index("c")
    sid = lax.axis_index("s")
    wid = sid * NC + cid
    pbase = wid * PPW
    pltpu.sync_copy(ux_hbm.at[pl.ds(pbase, PPW)], ux_v)
    pltpu.sync_copy(uy_hbm.at[pl.ds(pbase, PPW)], uy_v)

    def build_and_fire(bset, base, lvl):
        i00, i01, i10, i11, wx, wy, t00, t01, t10, t11, sem = bset
        off_y, w = LEVELS[lvl]

        @plsc.parallel_loop(0, G)
        def build_g(g):
            gx = ux_v[pl.ds(base + g * L, L)]
            gy = uy_v[pl.ds(base + g * L, L)]
            x = ((gx + 1.0) * w - 1.0) * 0.5
            y = ((gy + 1.0) * w - 1.0) * 0.5
            x = jnp.minimum(jnp.maximum(x, 0.0), w - 1.0)
            y = jnp.minimum(jnp.maximum(y, 0.0), w - 1.0)
            xi = x.astype(jnp.int32)
            yi = y.astype(jnp.int32)
            gs = pl.ds(g * L, L)
            b = (yi + off_y) * TW + xi
            i00[gs] = b
            i01[gs] = b + 1
            i10[gs] = b + TW
            i11[gs] = b + (TW + 1)
            wx[gs] = x - xi.astype(jnp.float32)
            wy[gs] = y - yi.astype(jnp.float32)

        pltpu.async_copy(table_hbm.at[i00], t00, sem)
        pltpu.async_copy(table_hbm.at[i01], t01, sem)
        pltpu.async_copy(table_hbm.at[i10], t10, sem)
        pltpu.async_copy(table_hbm.at[i11], t11, sem)

    def wait_and_combine(bset, lvl):
        i00, i01, i10, i11, wx, wy, t00, t01, t10, t11, sem = bset
        pltpu.make_async_copy(table_hbm.at[i00], t00, sem).wait()
        pltpu.make_async_copy(table_hbm.at[i01], t01, sem).wait()
        pltpu.make_async_copy(table_hbm.at[i10], t10, sem).wait()
        pltpu.make_async_copy(table_hbm.at[i11], t11, sem).wait()

        @plsc.parallel_loop(0, G)
        def comb_g(g):
            gs = pl.ds(g * L, L)
            wxv = wx[gs]
            wyv = wy[gs]

            def bcast(wv, i):
                s = lax.squeeze(lax.slice(wv, (i,), (i + 1,)), (0,))
                return jnp.broadcast_to(s, (L,))

            for i in range(L):
                pix = g * L + i
                fx = bcast(wxv, i)
                fy = bcast(wyv, i)
                a00 = t00[pix]
                a01 = t01[pix]
                a10 = t10[pix]
                a11 = t11[pix]
                top = a00 + fx * (a01 - a00)
                bot = a10 + fx * (a11 - a10)
                a = top + fy * (bot - top)
                if lvl == 0:
                    acc_v[pix] = a
                else:
                    plsc.addupdate(acc_v.at[pix], a)

    build_and_fire(bufs[0], 0, 0)
    build_and_fire(bufs[1], 0, 1)
    build_and_fire(bufs[2], 0, 2)

    def chunk_body(ch, carry):
        cbase = ch * CH
        for lvl in range(4):
            if lvl == 0:
                build_and_fire(bufs[3], cbase, 3)
            else:
                @pl.when(ch < NCHUNK - 1)
                def _prefetch():
                    build_and_fire(bufs[lvl - 1], cbase + CH, lvl - 1)
            wait_and_combine(bufs[lvl], lvl)
        pltpu.sync_copy(acc_v, out_hbm.at[pl.ds(pbase + cbase, CH)])
        return carry

    lax.fori_loop(0, NCHUNK, chunk_body, 0)


def kernel(uv_inputs, texture_id, data):
    tex = lax.dynamic_index_in_dim(data, texture_id, axis=0, keepdims=False)
    table = jnp.transpose(tex, (1, 2, 0)).reshape(TH * TW, C)
    ux = uv_inputs[0, 0].reshape(P)
    uy = uv_inputs[0, 1].reshape(P)

    mesh = plsc.VectorSubcoreMesh(core_axis_name="c", subcore_axis_name="s")
    bset = [
        pltpu.VMEM((CH,), jnp.int32),
        pltpu.VMEM((CH,), jnp.int32),
        pltpu.VMEM((CH,), jnp.int32),
        pltpu.VMEM((CH,), jnp.int32),
        pltpu.VMEM((CH,), jnp.float32),
        pltpu.VMEM((CH,), jnp.float32),
        pltpu.VMEM((CH, C), jnp.float32),
        pltpu.VMEM((CH, C), jnp.float32),
        pltpu.VMEM((CH, C), jnp.float32),
        pltpu.VMEM((CH, C), jnp.float32),
        pltpu.SemaphoreType.DMA,
    ]
    run = pl.kernel(
        _body,
        out_type=jax.ShapeDtypeStruct((P, C), jnp.float32),
        mesh=mesh,
        scratch_types=(
            [pltpu.VMEM((PPW,), jnp.float32), pltpu.VMEM((PPW,), jnp.float32)]
            + bset * 4
            + [pltpu.VMEM((CH, C), jnp.float32)]
        ),
        compiler_params=pltpu.CompilerParams(use_tc_tiling_on_sc=False),
    )
    out = run(ux, uy, table)
    return jnp.transpose(out.reshape(1, H, W, C), (0, 3, 1, 2))
